# Initial kernel scaffold; baseline (speedup 1.0000x reference)
import jax
import jax.numpy as jnp
from jax.experimental import pallas as pl


def kernel(art_embed, lab_embed, artist_idx, label_idx, noise_idxs):
    raise NotImplementedError("write your pallas kernel here")


# speedup vs baseline: 1.2190x; 1.2190x over previous
import functools

import jax
import jax.numpy as jnp
from jax import lax
from jax.experimental import pallas as pl
from jax.experimental.pallas import tpu as pltpu
from jax.experimental.pallas import tpu_sc as plsc

_CHUNK = 128
_NBUF = 7


def _sc_gather(art_embed, lab_embed, art_idx2d, lab_idx2d, n_art, n_lab, d):
    nw = 32
    art_rows_w = art_idx2d.shape[1]
    lab_rows_w = lab_idx2d.shape[1]

    mesh = plsc.VectorSubcoreMesh(core_axis_name="c", subcore_axis_name="s")

    @functools.partial(
        pl.kernel,
        out_type=[
            jax.ShapeDtypeStruct((n_art, d), jnp.float32),
            jax.ShapeDtypeStruct((n_lab, d), jnp.float32),
        ],
        mesh=mesh,
        compiler_params=pltpu.CompilerParams(use_tc_tiling_on_sc=False),
        scratch_types=[
            pltpu.VMEM((art_rows_w, _CHUNK), jnp.int32),
            pltpu.VMEM((lab_rows_w, _CHUNK), jnp.int32),
            pltpu.VMEM((_NBUF, _CHUNK, d), jnp.float32),
            pltpu.SemaphoreType.DMA,
            pltpu.SemaphoreType.DMA,
        ],
    )
    def sc_kernel(art_hbm, lab_hbm, aidx_hbm, lidx_hbm, art_out, lab_out,
                  aidx_v, lidx_v, rows_v, gsem, wsem):
        wid = lax.axis_index("s") * 2 + lax.axis_index("c")

        pltpu.sync_copy(aidx_hbm.at[wid], aidx_v)
        pltpu.sync_copy(lidx_hbm.at[wid], lidx_v)

        lab_base = wid * lab_rows_w * _CHUNK
        art_base = wid * art_rows_w * _CHUNK

        @pl.loop(0, lab_rows_w, step=_NBUF)
        def _(c0):
            gathers = [
                pltpu.async_copy(lab_hbm.at[lidx_v.at[c0 + b]], rows_v.at[b],
                                 gsem)
                for b in range(_NBUF)
            ]
            for cp in gathers:
                cp.wait()
            writes = [
                pltpu.async_copy(
                    rows_v.at[b],
                    lab_out.at[pl.ds(lab_base + (c0 + b) * _CHUNK, _CHUNK)],
                    wsem)
                for b in range(_NBUF)
            ]
            for cp in writes:
                cp.wait()

        gathers = [
            pltpu.async_copy(art_hbm.at[aidx_v.at[b]], rows_v.at[b], gsem)
            for b in range(art_rows_w)
        ]
        for cp in gathers:
            cp.wait()
        writes = [
            pltpu.async_copy(rows_v.at[b],
                             art_out.at[pl.ds(art_base + b * _CHUNK, _CHUNK)],
                             wsem)
            for b in range(art_rows_w)
        ]
        for cp in writes:
            cp.wait()

    return sc_kernel(art_embed, lab_embed, art_idx2d, lab_idx2d)


def _tc_scores(art_rows, lab_rows, batch, k, d, bb):
    nb = batch // bb

    def body(a_ref, g_ref, dots_ref, norm_ref):
        i = pl.program_id(0)
        j = pl.program_id(1)
        a = a_ref[...]
        g = g_ref[...]
        dots_ref[...] = jnp.sum(a * g, axis=1, keepdims=True)

        @pl.when(jnp.logical_and(i == 0, j == 0))
        def _():
            norm_ref[...] = jnp.zeros_like(norm_ref)

        @pl.when(j == 0)
        def _():
            norm_ref[...] += jnp.reshape(jnp.sum(a * a) + jnp.sum(g * g),
                                         (1, 1))

    dots, norm = pl.pallas_call(
        body,
        grid=(nb, k),
        in_specs=[
            pl.BlockSpec((bb, d), lambda i, j: (i, 0)),
            pl.BlockSpec((bb, d), lambda i, j: (j * nb + i, 0)),
        ],
        out_specs=[
            pl.BlockSpec((bb, 1), lambda i, j: (j * nb + i, 0)),
            pl.BlockSpec((1, 1), lambda i, j: (0, 0)),
        ],
        out_shape=[
            jax.ShapeDtypeStruct((k * batch, 1), jnp.float32),
            jax.ShapeDtypeStruct((1, 1), jnp.float32),
        ],
    )(art_rows, lab_rows)
    return dots, norm


def kernel(art_embed, lab_embed, artist_idx, label_idx, noise_idxs):
    batch = artist_idx.shape[0]
    d = art_embed.shape[1]
    n_neg = noise_idxs.shape[1]
    k = n_neg + 1

    lab_all = jnp.concatenate(
        [label_idx[None, :], noise_idxs.T.astype(jnp.int32)], axis=0)
    lab_idx2d = lab_all.reshape(32, -1, _CHUNK)
    art_idx2d = artist_idx.astype(jnp.int32).reshape(32, -1, _CHUNK)

    art_rows, lab_rows = _sc_gather(
        art_embed, lab_embed, art_idx2d, lab_idx2d,
        batch, k * batch, d)

    dots, norm = _tc_scores(art_rows, lab_rows, batch, k, d, bb=512)

    dots2 = dots.reshape(k, batch)
    scores = dots2[0][:, None]
    noise_scores = dots2[1:].T
    embed_norm = norm[0, 0] / jnp.float32(batch * d)
    return scores, noise_scores, embed_norm


# speedup vs baseline: 1.6583x; 1.3603x over previous
import functools

import jax
import jax.numpy as jnp
from jax import lax
from jax.experimental import pallas as pl
from jax.experimental.pallas import tpu as pltpu
from jax.experimental.pallas import tpu_sc as plsc

_CHUNK = 128
_NBUF = 7


def _sc_gather(art_embed, lab_embed, art_idx2d, lab_idx2d, n_art, n_lab, d):
    nw = 32
    art_rows_w = art_idx2d.shape[1]
    lab_rows_w = lab_idx2d.shape[1]

    mesh = plsc.VectorSubcoreMesh(core_axis_name="c", subcore_axis_name="s")

    @functools.partial(
        pl.kernel,
        out_type=[
            jax.ShapeDtypeStruct((n_art, d), jnp.float32),
            jax.ShapeDtypeStruct((n_lab, d), jnp.float32),
        ],
        mesh=mesh,
        compiler_params=pltpu.CompilerParams(use_tc_tiling_on_sc=False),
        scratch_types=[
            pltpu.VMEM((art_rows_w, _CHUNK), jnp.int32),
            pltpu.VMEM((lab_rows_w, _CHUNK), jnp.int32),
            pltpu.VMEM((_NBUF, _CHUNK, d), jnp.float32),
            pltpu.SemaphoreType.DMA,
            pltpu.SemaphoreType.DMA,
        ],
    )
    def sc_kernel(art_hbm, lab_hbm, aidx_hbm, lidx_hbm, art_out, lab_out,
                  aidx_v, lidx_v, rows_v, gsem, wsem):
        wid = lax.axis_index("s") * 2 + lax.axis_index("c")

        pltpu.sync_copy(aidx_hbm.at[wid], aidx_v)
        pltpu.sync_copy(lidx_hbm.at[wid], lidx_v)

        lab_base = wid * lab_rows_w * _CHUNK
        art_base = wid * art_rows_w * _CHUNK

        @pl.loop(0, lab_rows_w, step=_NBUF)
        def _(c0):
            gathers = [
                pltpu.async_copy(lab_hbm.at[lidx_v.at[c0 + b]], rows_v.at[b],
                                 gsem)
                for b in range(_NBUF)
            ]
            for cp in gathers:
                cp.wait()
            writes = [
                pltpu.async_copy(
                    rows_v.at[b],
                    lab_out.at[pl.ds(lab_base + (c0 + b) * _CHUNK, _CHUNK)],
                    wsem)
                for b in range(_NBUF)
            ]
            for cp in writes:
                cp.wait()

        gathers = [
            pltpu.async_copy(art_hbm.at[aidx_v.at[b]], rows_v.at[b], gsem)
            for b in range(art_rows_w)
        ]
        for cp in gathers:
            cp.wait()
        writes = [
            pltpu.async_copy(rows_v.at[b],
                             art_out.at[pl.ds(art_base + b * _CHUNK, _CHUNK)],
                             wsem)
            for b in range(art_rows_w)
        ]
        for cp in writes:
            cp.wait()

    return sc_kernel(art_embed, lab_embed, art_idx2d, lab_idx2d)


def _tc_scores(art_rows, lab_rows, batch, k, d, bb):
    nb = batch // bb

    rows_o = bb // 128

    def body(a_ref, g_ref, dots_ref):
        a = a_ref[...]
        g = g_ref[...]
        p = a * g
        ones_row = jnp.ones((1, d), dtype=jnp.float32)
        s = jax.lax.dot_general(ones_row, p, (((1,), (1,)), ((), ())),
                                preferred_element_type=jnp.float32)
        dots_ref[...] = s.reshape(rows_o, 128)

    dots = pl.pallas_call(
        body,
        grid=(nb, k),
        in_specs=[
            pl.BlockSpec((bb, d), lambda i, j: (i, 0)),
            pl.BlockSpec((bb, d), lambda i, j: (j * nb + i, 0)),
        ],
        out_specs=pl.BlockSpec((rows_o, 128), lambda i, j: (j * nb + i, 0)),
        out_shape=jax.ShapeDtypeStruct((k * batch // 128, 128), jnp.float32),
    )(art_rows, lab_rows)

    def norm_body(a_ref, g_ref, norm_ref):
        @pl.when(pl.program_id(0) == 0)
        def _():
            norm_ref[...] = jnp.zeros_like(norm_ref)

        a = a_ref[...]
        g = g_ref[...]
        norm_ref[...] += jnp.reshape(jnp.sum(a * a) + jnp.sum(g * g), (1, 1))

    norm = pl.pallas_call(
        norm_body,
        grid=(nb,),
        in_specs=[
            pl.BlockSpec((bb, d), lambda i: (i, 0)),
            pl.BlockSpec((bb, d), lambda i: (i, 0)),
        ],
        out_specs=pl.BlockSpec((1, 1), lambda i: (0, 0)),
        out_shape=jax.ShapeDtypeStruct((1, 1), jnp.float32),
    )(art_rows, lab_rows)
    return dots, norm


def kernel(art_embed, lab_embed, artist_idx, label_idx, noise_idxs):
    batch = artist_idx.shape[0]
    d = art_embed.shape[1]
    n_neg = noise_idxs.shape[1]
    k = n_neg + 1

    lab_all = jnp.concatenate(
        [label_idx[None, :], noise_idxs.T.astype(jnp.int32)], axis=0)
    lab_idx2d = lab_all.reshape(32, -1, _CHUNK)
    art_idx2d = artist_idx.astype(jnp.int32).reshape(32, -1, _CHUNK)

    art_rows, lab_rows = _sc_gather(
        art_embed, lab_embed, art_idx2d, lab_idx2d,
        batch, k * batch, d)

    dots, norm = _tc_scores(art_rows, lab_rows, batch, k, d, bb=2048)

    dots2 = dots.reshape(k, batch)
    scores = dots2[0][:, None]
    noise_scores = dots2[1:].T
    embed_norm = norm[0, 0] / jnp.float32(batch * d)
    return scores, noise_scores, embed_norm


# speedup vs baseline: 1.9093x; 1.1514x over previous
import functools

import jax
import jax.numpy as jnp
from jax import lax
from jax.experimental import pallas as pl
from jax.experimental.pallas import tpu as pltpu
from jax.experimental.pallas import tpu_sc as plsc

_CHUNK = 128
_NBUF = 6
_DP = 128


def _sc_gather(art_p, lab_p, art_idx3d, lab_idx3d, n_art, n_lab):
    art_rows_w = art_idx3d.shape[1]
    lab_rows_w = lab_idx3d.shape[1]

    mesh = plsc.VectorSubcoreMesh(core_axis_name="c", subcore_axis_name="s")

    @functools.partial(
        pl.kernel,
        out_type=[
            jax.ShapeDtypeStruct((n_art, _DP), jnp.float32),
            jax.ShapeDtypeStruct((n_lab, _DP), jnp.float32),
        ],
        mesh=mesh,
        compiler_params=pltpu.CompilerParams(use_tc_tiling_on_sc=False),
        scratch_types=[
            pltpu.VMEM((art_rows_w, _CHUNK), jnp.int32),
            pltpu.VMEM((lab_rows_w, _CHUNK), jnp.int32),
            pltpu.VMEM((_NBUF, _CHUNK, _DP), jnp.float32),
            pltpu.SemaphoreType.DMA,
            pltpu.SemaphoreType.DMA,
        ],
    )
    def sc_kernel(art_hbm, lab_hbm, aidx_hbm, lidx_hbm, art_out, lab_out,
                  aidx_v, lidx_v, rows_v, gsem, wsem):
        wid = lax.axis_index("s") * 2 + lax.axis_index("c")

        pltpu.sync_copy(aidx_hbm.at[wid], aidx_v)
        pltpu.sync_copy(lidx_hbm.at[wid], lidx_v)

        lab_base = wid * lab_rows_w * _CHUNK
        art_base = wid * art_rows_w * _CHUNK

        @pl.loop(0, lab_rows_w, step=_NBUF)
        def _(c0):
            gathers = [
                pltpu.async_copy(lab_hbm.at[lidx_v.at[c0 + b]], rows_v.at[b],
                                 gsem)
                for b in range(_NBUF)
            ]
            for cp in gathers:
                cp.wait()
            writes = [
                pltpu.async_copy(
                    rows_v.at[b],
                    lab_out.at[pl.ds(lab_base + (c0 + b) * _CHUNK, _CHUNK)],
                    wsem)
                for b in range(_NBUF)
            ]
            for cp in writes:
                cp.wait()

        gathers = [
            pltpu.async_copy(art_hbm.at[aidx_v.at[b]], rows_v.at[b], gsem)
            for b in range(art_rows_w)
        ]
        for cp in gathers:
            cp.wait()
        writes = [
            pltpu.async_copy(rows_v.at[b],
                             art_out.at[pl.ds(art_base + b * _CHUNK, _CHUNK)],
                             wsem)
            for b in range(art_rows_w)
        ]
        for cp in writes:
            cp.wait()

    return sc_kernel(art_p, lab_p, art_idx3d, lab_idx3d)


def _tc_scores(art_rows, lab_rows, batch, k, bb):
    nb = batch // bb
    rows_o = bb // 128

    def body(a_ref, g_ref, dots_ref):
        a = a_ref[...]
        g = g_ref[...]
        p = a * g
        ones_row = jnp.ones((1, _DP), dtype=jnp.float32)
        s = jax.lax.dot_general(ones_row, p, (((1,), (1,)), ((), ())),
                                preferred_element_type=jnp.float32)
        dots_ref[...] = s.reshape(rows_o, 128)

    dots = pl.pallas_call(
        body,
        grid=(nb, k),
        in_specs=[
            pl.BlockSpec((bb, _DP), lambda i, j: (i, 0)),
            pl.BlockSpec((bb, _DP), lambda i, j: (j * nb + i, 0)),
        ],
        out_specs=pl.BlockSpec((rows_o, 128), lambda i, j: (j * nb + i, 0)),
        out_shape=jax.ShapeDtypeStruct((k * batch // 128, 128), jnp.float32),
    )(art_rows, lab_rows)

    def norm_body(a_ref, g_ref, norm_ref):
        @pl.when(pl.program_id(0) == 0)
        def _():
            norm_ref[...] = jnp.zeros_like(norm_ref)

        a = a_ref[...]
        g = g_ref[...]
        norm_ref[...] += jnp.reshape(jnp.sum(a * a) + jnp.sum(g * g), (1, 1))

    norm = pl.pallas_call(
        norm_body,
        grid=(nb,),
        in_specs=[
            pl.BlockSpec((bb, _DP), lambda i: (i, 0)),
            pl.BlockSpec((bb, _DP), lambda i: (i, 0)),
        ],
        out_specs=pl.BlockSpec((1, 1), lambda i: (0, 0)),
        out_shape=jax.ShapeDtypeStruct((1, 1), jnp.float32),
    )(art_rows, lab_rows)
    return dots, norm


def kernel(art_embed, lab_embed, artist_idx, label_idx, noise_idxs):
    batch = artist_idx.shape[0]
    d = art_embed.shape[1]
    n_neg = noise_idxs.shape[1]
    k = n_neg + 1

    art_p = jnp.pad(art_embed, ((0, 0), (0, _DP - d)))
    lab_p = jnp.pad(lab_embed, ((0, 0), (0, _DP - d)))

    lab_all = jnp.concatenate(
        [label_idx[None, :], noise_idxs.T.astype(jnp.int32)], axis=0)
    lab_idx3d = lab_all.reshape(32, -1, _CHUNK)
    art_idx3d = artist_idx.astype(jnp.int32).reshape(32, -1, _CHUNK)

    art_rows, lab_rows = _sc_gather(
        art_p, lab_p, art_idx3d, lab_idx3d, batch, k * batch)

    dots, norm = _tc_scores(art_rows, lab_rows, batch, k, bb=2048)

    dots2 = dots.reshape(k, batch)
    scores = dots2[0][:, None]
    noise_scores = dots2[1:].T
    embed_norm = norm[0, 0] / jnp.float32(batch * d)
    return scores, noise_scores, embed_norm
